# Initial kernel scaffold; baseline (speedup 1.0000x reference)
import jax
import jax.numpy as jnp
from jax.experimental import pallas as pl


def kernel(x, token_emb, pos_emb):
    raise NotImplementedError("write your pallas kernel here")


# speedup vs baseline: 3.1271x; 3.1271x over previous
import functools

import jax
import jax.numpy as jnp
from jax import lax
from jax.experimental import pallas as pl
from jax.experimental.pallas import tpu as pltpu
from jax.experimental.pallas import tpu_sc as plsc

BATCH = 4096
MAXLEN = 200
EMBED = 32

_NC = 2
_NS = 16
_NW = _NC * _NS
_ROWS_PER_W = BATCH // _NW

_S0 = 128
_S1 = MAXLEN - _S0


def _emb_body(x_hbm, tok_hbm, pos_hbm, out_hbm, idx_v, rows_v, pos_v, sem):
    wid = lax.axis_index("s") * _NC + lax.axis_index("c")

    pltpu.sync_copy(pos_hbm, pos_v)

    def per_row(i, carry):
        b = wid * _ROWS_PER_W + i
        pltpu.sync_copy(x_hbm.at[b], idx_v)
        cp0 = pltpu.async_copy(
            tok_hbm.at[idx_v.at[pl.ds(0, _S0)]], rows_v.at[pl.ds(0, _S0)], sem
        )
        cp1 = pltpu.async_copy(
            tok_hbm.at[idx_v.at[pl.ds(_S0, _S1)]], rows_v.at[pl.ds(_S0, _S1)], sem
        )
        cp0.wait()
        cp1.wait()

        def add_t(t, c):
            rows_v[t, pl.ds(0, 16)] += pos_v[t, pl.ds(0, 16)]
            rows_v[t, pl.ds(16, 16)] += pos_v[t, pl.ds(16, 16)]
            return c

        lax.fori_loop(0, MAXLEN, add_t, 0, unroll=4)

        pltpu.sync_copy(rows_v, out_hbm.at[b])
        return carry

    lax.fori_loop(0, _ROWS_PER_W, per_row, 0)


@functools.partial(jax.jit, static_argnums=())
def _emb_call(x, token_emb, pos_emb):
    mesh = plsc.VectorSubcoreMesh(core_axis_name="c", subcore_axis_name="s")
    k = functools.partial(
        pl.kernel,
        mesh=mesh,
        out_type=jax.ShapeDtypeStruct((BATCH, MAXLEN, EMBED), jnp.float32),
        scratch_types=[
            pltpu.VMEM((MAXLEN,), jnp.int32),
            pltpu.VMEM((MAXLEN, EMBED), jnp.float32),
            pltpu.VMEM((MAXLEN, EMBED), jnp.float32),
            pltpu.SemaphoreType.DMA,
        ],
        compiler_params=pltpu.CompilerParams(use_tc_tiling_on_sc=False),
    )(_emb_body)
    return k(x, token_emb, pos_emb)


def kernel(x, token_emb, pos_emb):
    return _emb_call(x.astype(jnp.int32), token_emb, pos_emb)


# speedup vs baseline: 5.1930x; 1.6606x over previous
import functools

import jax
import jax.numpy as jnp
from jax import lax
from jax.experimental import pallas as pl
from jax.experimental.pallas import tpu as pltpu
from jax.experimental.pallas import tpu_sc as plsc

BATCH = 4096
MAXLEN = 200
EMBED = 32

_NC = 2
_NS = 16
_NW = _NC * _NS
_N_PER_W = BATCH * MAXLEN // _NW
_CH = 4 * MAXLEN
_NCH = _N_PER_W // _CH
_KB = _CH // MAXLEN

_G_SPLITS = [(0, 128), (128, 128), (256, 128), (384, 128), (512, 128),
             (640, 128), (768, 32)]


def _issue_gather(tok_hbm, idx_all, rows_b, sem, base):
    for s, sz in _G_SPLITS:
        pltpu.async_copy(
            tok_hbm.at[idx_all.at[pl.ds(base + s, sz)]],
            rows_b.at[pl.ds(s, sz)],
            sem,
        )


def _wait_gather(tok_hbm, rows_b, sem):
    pltpu.make_async_copy(tok_hbm.at[pl.ds(0, _CH)], rows_b, sem).wait()


def _add_pos(rows_b, pos_v):
    def add_t(t, c):
        p0 = pos_v[t, pl.ds(0, 16)]
        p1 = pos_v[t, pl.ds(16, 16)]
        for q in range(_KB):
            rows_b[q * MAXLEN + t, pl.ds(0, 16)] += p0
            rows_b[q * MAXLEN + t, pl.ds(16, 16)] += p1
        return c

    lax.fori_loop(0, MAXLEN, add_t, 0, unroll=2)


def _emb_body(x_hbm, tok_hbm, pos_hbm, out_hbm,
              idx_all, pos_v, rows2, gsem0, gsem1, osem0, osem1):
    wid = lax.axis_index("s") * _NC + lax.axis_index("c")
    wbase = wid * _N_PER_W

    buf0 = rows2.at[0]
    buf1 = rows2.at[1]

    pltpu.sync_copy(pos_hbm, pos_v)
    pltpu.sync_copy(x_hbm.at[pl.ds(wbase, _N_PER_W)], idx_all)

    _issue_gather(tok_hbm, idx_all, buf0, gsem0, 0)

    def outer(j, carry):
        ca = 2 * j
        cb = 2 * j + 1

        _wait_gather(tok_hbm, buf0, gsem0)

        @pl.when(j > 0)
        def _():
            pltpu.make_async_copy(buf1, out_hbm.at[pl.ds(0, _CH)], osem1).wait()

        _issue_gather(tok_hbm, idx_all, buf1, gsem1, cb * _CH)
        _add_pos(buf0, pos_v)
        pltpu.async_copy(buf0, out_hbm.at[pl.ds(wbase + ca * _CH, _CH)], osem0)

        _wait_gather(tok_hbm, buf1, gsem1)
        pltpu.make_async_copy(buf0, out_hbm.at[pl.ds(0, _CH)], osem0).wait()

        @pl.when(j < _NCH // 2 - 1)
        def _():
            _issue_gather(tok_hbm, idx_all, buf0, gsem0, (ca + 2) * _CH)

        _add_pos(buf1, pos_v)
        pltpu.async_copy(buf1, out_hbm.at[pl.ds(wbase + cb * _CH, _CH)], osem1)
        return carry

    lax.fori_loop(0, _NCH // 2, outer, 0)

    pltpu.make_async_copy(buf1, out_hbm.at[pl.ds(0, _CH)], osem1).wait()


@jax.jit
def _emb_call(x_flat, token_emb, pos_emb):
    mesh = plsc.VectorSubcoreMesh(core_axis_name="c", subcore_axis_name="s")
    k = functools.partial(
        pl.kernel,
        mesh=mesh,
        out_type=jax.ShapeDtypeStruct((BATCH * MAXLEN, EMBED), jnp.float32),
        scratch_types=[
            pltpu.VMEM((_N_PER_W,), jnp.int32),
            pltpu.VMEM((MAXLEN, EMBED), jnp.float32),
            pltpu.VMEM((2, _CH, EMBED), jnp.float32),
            pltpu.SemaphoreType.DMA,
            pltpu.SemaphoreType.DMA,
            pltpu.SemaphoreType.DMA,
            pltpu.SemaphoreType.DMA,
        ],
        compiler_params=pltpu.CompilerParams(use_tc_tiling_on_sc=False),
    )(_emb_body)
    return k(x_flat, token_emb, pos_emb)


def kernel(x, token_emb, pos_emb):
    out = _emb_call(x.reshape(-1).astype(jnp.int32), token_emb, pos_emb)
    return out.reshape(BATCH, MAXLEN, EMBED)
